# Initial kernel scaffold; baseline (speedup 1.0000x reference)
import jax
import jax.numpy as jnp
from jax.experimental import pallas as pl


def kernel(speaker_labels, table):
    raise NotImplementedError("write your pallas kernel here")


# speedup vs baseline: 1.1804x; 1.1804x over previous
import functools

import jax
import jax.numpy as jnp
from jax import lax
from jax.experimental import pallas as pl
from jax.experimental.pallas import tpu as pltpu
from jax.experimental.pallas import tpu_sc as plsc

B, S, D = 16384, 200, 64
N = B * S
NP = N // 2
NC, NS = 2, 16
NW = NC * NS
PPW = NP // NW
CHUNK = 512
NCH = PPW // CHUNK
SUB = 128
NSUB = CHUNK // SUB


def _build():
    mesh = plsc.VectorSubcoreMesh(core_axis_name="c", subcore_axis_name="s")

    @functools.partial(
        pl.kernel,
        mesh=mesh,
        out_type=jax.ShapeDtypeStruct((NP, 2 * D), jnp.float32),
        scratch_types=[
            pltpu.VMEM((CHUNK,), jnp.int32),
            pltpu.VMEM((CHUNK, 2 * D), jnp.float32),
            pltpu.SemaphoreType.DMA,
        ],
    )
    def lookup(ptab_hbm, pidx_hbm, out_hbm, idx_v, rows_v, sem):
        wid = lax.axis_index("s") * NC + lax.axis_index("c")
        wbase = wid * PPW

        def chunk_body(i, carry):
            base = pl.multiple_of(wbase + i * CHUNK, CHUNK)
            pltpu.sync_copy(pidx_hbm.at[pl.ds(base, CHUNK)], idx_v)
            handles = [
                pltpu.async_copy(
                    ptab_hbm.at[idx_v.at[pl.ds(j * SUB, SUB)]],
                    rows_v.at[pl.ds(j * SUB, SUB)],
                    sem,
                )
                for j in range(NSUB)
            ]
            for h in handles:
                h.wait()
            pltpu.sync_copy(rows_v, out_hbm.at[pl.ds(base, CHUNK)])
            return carry

        lax.fori_loop(0, NCH, chunk_body, 0)

    return lookup


_lookup = _build()


@jax.jit
def kernel(speaker_labels, table):
    pairs = speaker_labels.reshape(NP, 2)
    pidx = 3 * pairs[:, 0] + pairs[:, 1]
    ptab = jnp.concatenate(
        [jnp.repeat(table, 3, axis=0), jnp.tile(table, (3, 1))], axis=1
    )
    out = _lookup(ptab, pidx)
    return out.reshape(B, S, D)


# speedup vs baseline: 9.4289x; 7.9881x over previous
import functools

import jax
import jax.numpy as jnp
from jax import lax
from jax.experimental import pallas as pl
from jax.experimental.pallas import tpu as pltpu
from jax.experimental.pallas import tpu_sc as plsc

B, S, D = 16384, 200, 64
N = B * S
NC, NS = 2, 16
NW = NC * NS
BPW = N // NW
CHUNK = 400
NCH = BPW // CHUNK
NG = CHUNK // 16

_GDN = lax.GatherDimensionNumbers(
    offset_dims=(), collapsed_slice_dims=(0,), start_index_map=(0,)
)


def _splat(vec, lane):
    idx = jnp.full((16, 1), lane, dtype=jnp.int32)
    return lax.gather(
        vec, idx, _GDN, (1,), mode=lax.GatherScatterMode.PROMISE_IN_BOUNDS
    )


def _build():
    mesh = plsc.VectorSubcoreMesh(core_axis_name="c", subcore_axis_name="s")

    @functools.partial(
        pl.kernel,
        mesh=mesh,
        out_type=jax.ShapeDtypeStruct((N, D), jnp.float32),
        scratch_types=[
            pltpu.VMEM((192,), jnp.float32),
            pltpu.VMEM((CHUNK,), jnp.int32),
            pltpu.VMEM((CHUNK, D), jnp.float32),
            pltpu.VMEM((CHUNK, D), jnp.float32),
            pltpu.SemaphoreType.DMA,
            pltpu.SemaphoreType.DMA,
        ],
    )
    def lookup(tab_hbm, idx_hbm, out_hbm, tab_v, idx_v, rows_a, rows_b, sem_a, sem_b):
        wid = lax.axis_index("s") * NC + lax.axis_index("c")
        wbase = wid * BPW
        pltpu.sync_copy(tab_hbm, tab_v)
        trow = [
            [tab_v[pl.ds(r * D + db * 16, 16)] for db in range(D // 16)]
            for r in range(3)
        ]
        d1 = [trow[1][db] - trow[0][db] for db in range(D // 16)]
        d2 = [trow[2][db] - trow[0][db] for db in range(D // 16)]

        def fill(rows_v, base):
            pltpu.sync_copy(idx_hbm.at[pl.ds(base, CHUNK)], idx_v)

            def group(g, carry):
                ivec = idx_v[pl.ds(g * 16, 16)]
                for l in range(16):
                    lab = _splat(ivec, l)
                    f1 = (lab & 1).astype(jnp.float32)
                    f2 = (lab >> 1).astype(jnp.float32)
                    for db in range(D // 16):
                        v = trow[0][db] + f1 * d1[db] + f2 * d2[db]
                        rows_v[g * 16 + l, pl.ds(db * 16, 16)] = v
                return carry

            lax.fori_loop(0, NG, group, 0)

        def chunk_pair(i, carry):
            base_a = pl.multiple_of(wbase + (2 * i) * CHUNK, CHUNK)
            base_b = pl.multiple_of(wbase + (2 * i + 1) * CHUNK, CHUNK)

            @pl.when(i > 0)
            def _():
                pltpu.make_async_copy(
                    rows_a, out_hbm.at[pl.ds(0, CHUNK)], sem_a
                ).wait()

            fill(rows_a, base_a)
            pltpu.async_copy(rows_a, out_hbm.at[pl.ds(base_a, CHUNK)], sem_a)

            @pl.when(i > 0)
            def _():
                pltpu.make_async_copy(
                    rows_b, out_hbm.at[pl.ds(0, CHUNK)], sem_b
                ).wait()

            fill(rows_b, base_b)
            pltpu.async_copy(rows_b, out_hbm.at[pl.ds(base_b, CHUNK)], sem_b)
            return carry

        lax.fori_loop(0, NCH // 2, chunk_pair, 0)
        pltpu.make_async_copy(rows_a, out_hbm.at[pl.ds(0, CHUNK)], sem_a).wait()
        pltpu.make_async_copy(rows_b, out_hbm.at[pl.ds(0, CHUNK)], sem_b).wait()

    return lookup


_lookup = _build()


@jax.jit
def kernel(speaker_labels, table):
    idx = speaker_labels.reshape(N)
    tab = table.reshape(3 * D)
    out = _lookup(tab, idx)
    return out.reshape(B, S, D)
